# baseline (speedup 1.0000x reference)
import functools

import jax
import jax.numpy as jnp
from jax import lax
from jax.experimental import pallas as pl
from jax.experimental.pallas import tpu as pltpu
from jax.experimental.pallas import tpu_sc as plsc

B = 1024
N = 20000
ROWS, COLS = 82, 67
NCELL = ROWS * COLS
CP = 5504
LANES = 16
NW = 32
B_PER_W = B // NW
STAGE = 2000
N_STAGES = N // STAGE
CHUNKS_PER_STAGE = STAGE // LANES
GATHER_CHUNKS = CP // LANES
NSLOTS = 4


def _regrid_sc(inputs, row_indices, col_indices):
    mesh = plsc.VectorSubcoreMesh(core_axis_name="c", subcore_axis_name="s")

    @functools.partial(
        pl.kernel,
        mesh=mesh,
        out_type=jax.ShapeDtypeStruct((B, CP), jnp.float32),
        compiler_params=pltpu.CompilerParams(needs_layout_passes=False),
        scratch_types=[
            pltpu.VMEM((CP,), jnp.int32),
            pltpu.VMEM((CP + LANES,), jnp.int32),
            pltpu.VMEM((STAGE,), jnp.int32),
            pltpu.VMEM((STAGE,), jnp.int32),
            pltpu.VMEM((N,), jnp.float32),
            pltpu.VMEM((N,), jnp.float32),
            pltpu.VMEM((N,), jnp.float32),
            pltpu.VMEM((N,), jnp.float32),
            pltpu.VMEM((CP,), jnp.float32),
            pltpu.VMEM((CP,), jnp.float32),
            pltpu.VMEM((CP,), jnp.float32),
            pltpu.VMEM((CP,), jnp.float32),
            pltpu.SemaphoreType.DMA,
            pltpu.SemaphoreType.DMA,
            pltpu.SemaphoreType.DMA,
            pltpu.SemaphoreType.DMA,
            pltpu.SemaphoreType.DMA,
            pltpu.SemaphoreType.DMA,
            pltpu.SemaphoreType.DMA,
            pltpu.SemaphoreType.DMA,
        ],
    )
    def k(in_hbm, row_hbm, col_hbm, out_hbm,
          winner, empty, rbuf, cbuf, inA, inB, inC, inD,
          outA, outB, outC, outD,
          isem0, isem1, isem2, isem3, osem0, osem1, osem2, osem3):
        lane = lax.iota(jnp.int32, LANES)
        wid = lax.axis_index("c") * 16 + lax.axis_index("s")
        base = wid * B_PER_W
        ins = (inA, inB, inC, inD)
        outs = (outA, outB, outC, outD)
        isems = (isem0, isem1, isem2, isem3)
        osems = (osem0, osem1, osem2, osem3)

        for s in range(NSLOTS):
            pltpu.async_copy(in_hbm.at[base + s], ins[s], isems[s])

        neg1 = jnp.full((LANES,), -1, jnp.int32)
        junk = jnp.full((LANES,), CP - 1, jnp.int32)
        zeros16 = jnp.zeros((LANES,), jnp.float32)

        def init_body(i, carry):
            winner[pl.ds(i * LANES, LANES)] = neg1
            empty[pl.ds(i * LANES, LANES)] = junk
            return carry

        lax.fori_loop(0, GATHER_CHUNKS, init_body, 0)
        empty[pl.ds(CP, LANES)] = junk

        for s in range(N_STAGES):
            pltpu.sync_copy(row_hbm.at[pl.ds(s * STAGE, STAGE)], rbuf)
            pltpu.sync_copy(col_hbm.at[pl.ds(s * STAGE, STAGE)], cbuf)

            def scat_body(j, carry, s=s):
                r = rbuf[pl.ds(j * LANES, LANES)]
                c = cbuf[pl.ds(j * LANES, LANES)]
                cell = r * COLS + c
                n_vec = (s * STAGE + j * LANES) + lane
                plsc.store_scatter(winner, [cell], n_vec)
                return carry

            lax.fori_loop(0, CHUNKS_PER_STAGE, scat_body, 0)

        def safe_body(j, cnt):
            sl = pl.ds(j * LANES, LANES)
            w = winner[sl]
            cells = j * LANES + lane
            is_empty = (w < 0) & (cells < NCELL)
            winner[sl] = jnp.maximum(w, jnp.zeros((LANES,), jnp.int32))
            plsc.store_compressed(
                empty.at[pl.ds(cnt, LANES)], cells, mask=is_empty)
            npop = plsc.all_reduce_population_count(is_empty)
            return cnt + lax.reduce_max(npop, axes=(0,))

        ecnt = lax.fori_loop(0, GATHER_CHUNKS, safe_body, 0)
        ezchunks = (ecnt + LANES - 1) // LANES

        @pl.loop(0, B_PER_W, step=NSLOTS)
        def quad(bl):
            for s in range(NSLOTS):
                rl = bl + s
                b = base + rl
                @pl.when(rl >= NSLOTS)
                def _():
                    pltpu.make_async_copy(
                        outs[s], out_hbm.at[b - NSLOTS], osems[s]).wait()
                pltpu.make_async_copy(in_hbm.at[b], ins[s], isems[s]).wait()

                def gat_body(j, carry, s=s):
                    sl = pl.ds(j * LANES, LANES)
                    outs[s][sl] = plsc.load_gather(ins[s], [winner[sl]])
                    return carry

                lax.fori_loop(0, GATHER_CHUNKS, gat_body, 0)

                def zero_body(j, carry, s=s):
                    idxs = empty[pl.ds(j * LANES, LANES)]
                    plsc.store_scatter(outs[s], [idxs], zeros16)
                    return carry

                lax.fori_loop(0, ezchunks, zero_body, 0)

                pltpu.async_copy(outs[s], out_hbm.at[b], osems[s])

                @pl.when(rl + NSLOTS < B_PER_W)
                def _():
                    pltpu.async_copy(in_hbm.at[b + NSLOTS], ins[s], isems[s])

        for s in range(NSLOTS):
            pltpu.make_async_copy(
                outs[s], out_hbm.at[base + B_PER_W - NSLOTS + s],
                osems[s]).wait()

    return k(inputs, row_indices, col_indices)


def kernel(inputs, row_indices, col_indices):
    out = _regrid_sc(inputs, row_indices, col_indices)
    return out[:, :NCELL].reshape(B, ROWS, COLS, 1)


# speedup vs baseline: 1.0413x; 1.0413x over previous
import functools

import jax
import jax.numpy as jnp
from jax import lax
from jax.experimental import pallas as pl
from jax.experimental.pallas import tpu as pltpu
from jax.experimental.pallas import tpu_sc as plsc

B = 1024
N = 20000
ROWS, COLS = 82, 67
NCELL = ROWS * COLS
CP = 5504
LANES = 16
NW = 32
B_PER_W = B // NW
STAGE = 2000
N_STAGES = N // STAGE
CHUNKS_PER_STAGE = STAGE // LANES
GATHER_CHUNKS = CP // LANES


def _regrid_sc(inputs, row_indices, col_indices):
    mesh = plsc.VectorSubcoreMesh(core_axis_name="c", subcore_axis_name="s")

    @functools.partial(
        pl.kernel,
        mesh=mesh,
        out_type=jax.ShapeDtypeStruct((B, CP), jnp.float32),
        compiler_params=pltpu.CompilerParams(needs_layout_passes=False),
        scratch_types=[
            pltpu.VMEM((CP,), jnp.int32),
            pltpu.VMEM((CP,), jnp.float32),
            pltpu.VMEM((STAGE,), jnp.int32),
            pltpu.VMEM((STAGE,), jnp.int32),
            pltpu.VMEM((N,), jnp.float32),
            pltpu.VMEM((N,), jnp.float32),
            pltpu.VMEM((CP,), jnp.float32),
            pltpu.VMEM((CP,), jnp.float32),
            pltpu.SemaphoreType.DMA,
            pltpu.SemaphoreType.DMA,
            pltpu.SemaphoreType.DMA,
            pltpu.SemaphoreType.DMA,
        ],
    )
    def k(in_hbm, row_hbm, col_hbm, out_hbm,
          winner, maskf, rbuf, cbuf, inA, inB, outA, outB,
          isem0, isem1, osem0, osem1):
        lane = lax.iota(jnp.int32, LANES)
        wid = lax.axis_index("c") * 16 + lax.axis_index("s")
        base = wid * B_PER_W
        ins = (inA, inB)
        outs = (outA, outB)
        isems = (isem0, isem1)
        osems = (osem0, osem1)

        for s in range(2):
            pltpu.async_copy(in_hbm.at[base + s], ins[s], isems[s])

        neg1 = jnp.full((LANES,), -1, jnp.int32)

        def init_body(i, carry):
            winner[pl.ds(i * LANES, LANES)] = neg1
            return carry

        lax.fori_loop(0, GATHER_CHUNKS, init_body, 0)

        for s in range(N_STAGES):
            pltpu.sync_copy(row_hbm.at[pl.ds(s * STAGE, STAGE)], rbuf)
            pltpu.sync_copy(col_hbm.at[pl.ds(s * STAGE, STAGE)], cbuf)

            def scat_body(j, carry, s=s):
                r = rbuf[pl.ds(j * LANES, LANES)]
                c = cbuf[pl.ds(j * LANES, LANES)]
                cell = r * COLS + c
                n_vec = (s * STAGE + j * LANES) + lane
                plsc.store_scatter(winner, [cell], n_vec)
                return carry

            lax.fori_loop(0, CHUNKS_PER_STAGE, scat_body, 0)

        def safe_body(j, carry):
            sl = pl.ds(j * LANES, LANES)
            w = winner[sl]
            filled = w >= 0
            winner[sl] = jnp.maximum(w, jnp.zeros((LANES,), jnp.int32))
            maskf[sl] = jnp.where(
                filled, jnp.full((LANES,), 1.0, jnp.float32),
                jnp.zeros((LANES,), jnp.float32))
            return carry

        lax.fori_loop(0, GATHER_CHUNKS, safe_body, 0)

        @pl.loop(0, B_PER_W, step=2)
        def pair(bl):
            for s in range(2):
                rl = bl + s
                b = base + rl
                @pl.when(rl >= 2)
                def _():
                    pltpu.make_async_copy(
                        outs[s], out_hbm.at[b - 2], osems[s]).wait()
                pltpu.make_async_copy(in_hbm.at[b], ins[s], isems[s]).wait()

                def gat_body(j, carry, s=s):
                    sl = pl.ds(j * LANES, LANES)
                    idx = winner[sl]
                    outs[s][sl] = plsc.load_gather(ins[s], [idx]) * maskf[sl]
                    return carry

                lax.fori_loop(0, GATHER_CHUNKS, gat_body, 0)

                pltpu.async_copy(outs[s], out_hbm.at[b], osems[s])

                @pl.when(rl + 2 < B_PER_W)
                def _():
                    pltpu.async_copy(in_hbm.at[b + 2], ins[s], isems[s])

        for s in range(2):
            pltpu.make_async_copy(
                outs[s], out_hbm.at[base + B_PER_W - 2 + s], osems[s]).wait()

    return k(inputs, row_indices, col_indices)


def kernel(inputs, row_indices, col_indices):
    out = _regrid_sc(inputs, row_indices, col_indices)
    return out[:, :NCELL].reshape(B, ROWS, COLS, 1)
